# Initial kernel scaffold; baseline (speedup 1.0000x reference)
import jax
import jax.numpy as jnp
from jax.experimental import pallas as pl


def kernel(x, W, b, Wr, br):
    raise NotImplementedError("write your pallas kernel here")


# speedup vs baseline: 2.6123x; 2.6123x over previous
import functools

import jax
import jax.numpy as jnp
from jax import lax
from jax.experimental import pallas as pl
from jax.experimental.pallas import tpu as pltpu

D_MODEL = 768
N_EXPERTS = 64
N_TOKENS = 2048
ROW_TILE = 64
N_BUF = 2560

_INTERPRET = False


def _router_body(x_ref, wr_ref, br_ref, logits_ref, w_ref, idx_ref):
    x = x_ref[...]
    logits = lax.dot_general(
        x, wr_ref[...], (((1,), (1,)), ((), ())),
        preferred_element_type=jnp.float32) + br_ref[...]
    logits_ref[...] = logits
    m = jnp.max(logits, axis=1, keepdims=True)
    p = jnp.exp(logits - m)
    s = jnp.sum(p, axis=1, keepdims=True)
    w_ref[...] = 1.0 / s
    lane = lax.broadcasted_iota(jnp.int32, (N_TOKENS, N_EXPERTS), 1)
    cand = jnp.where(logits == m, lane, N_EXPERTS)
    idx_ref[...] = jnp.min(cand, axis=1, keepdims=True)


def _router(x, Wr, br):
    return pl.pallas_call(
        _router_body,
        out_shape=[
            jax.ShapeDtypeStruct((N_TOKENS, N_EXPERTS), jnp.float32),
            jax.ShapeDtypeStruct((N_TOKENS, 1), jnp.float32),
            jax.ShapeDtypeStruct((N_TOKENS, 1), jnp.int32),
        ],
        interpret=_INTERPRET,
    )(x, Wr, br.reshape(1, N_EXPERTS))


def _expert_body(offs_ref, xs_ref, ws_ref, w_ref, b_ref, ys_ref):
    e = pl.program_id(0)
    start = offs_ref[e]
    stop = offs_ref[e + 1]
    n_tiles = (stop - start + ROW_TILE - 1) // ROW_TILE

    def tile(t, carry):
        s = pl.multiple_of(start + t * ROW_TILE, 8)
        xt = xs_ref[pl.ds(s, ROW_TILE), :]
        y = lax.dot_general(
            xt, w_ref[0], (((1,), (1,)), ((), ())),
            preferred_element_type=jnp.float32)
        y = (y + b_ref[0]) * ws_ref[pl.ds(s, ROW_TILE), :]
        ys_ref[pl.ds(s, ROW_TILE), :] = y
        return carry

    lax.fori_loop(0, n_tiles, tile, 0)


def _expert_matmul(offsets, x_sorted, w_sorted, W, b):
    grid_spec = pltpu.PrefetchScalarGridSpec(
        num_scalar_prefetch=1,
        grid=(N_EXPERTS,),
        in_specs=[
            pl.BlockSpec((N_BUF, D_MODEL), lambda e, offs: (0, 0)),
            pl.BlockSpec((N_BUF, 1), lambda e, offs: (0, 0)),
            pl.BlockSpec((1, D_MODEL, D_MODEL), lambda e, offs: (e, 0, 0)),
            pl.BlockSpec((1, 1, D_MODEL), lambda e, offs: (e, 0, 0)),
        ],
        out_specs=pl.BlockSpec((N_BUF, D_MODEL), lambda e, offs: (0, 0)),
    )
    return pl.pallas_call(
        _expert_body,
        grid_spec=grid_spec,
        out_shape=jax.ShapeDtypeStruct((N_BUF, D_MODEL), jnp.float32),
        interpret=_INTERPRET,
    )(offsets, x_sorted, w_sorted, W, b.reshape(N_EXPERTS, 1, D_MODEL))


def kernel(x, W, b, Wr, br):
    logits, w, idx = _router(x, Wr, br)
    idx_f = idx[:, 0]
    counts = jnp.zeros((N_EXPERTS,), jnp.int32).at[idx_f].add(1)
    padded = (counts + 7) & ~7
    offsets = jnp.concatenate(
        [jnp.zeros((1,), jnp.int32), jnp.cumsum(padded, dtype=jnp.int32)])
    start_unp = jnp.concatenate(
        [jnp.zeros((1,), jnp.int32), jnp.cumsum(counts, dtype=jnp.int32)])
    order = jnp.argsort(idx_f, stable=True)
    e_sorted = idx_f[order]
    pos_sorted = offsets[e_sorted] + (
        jnp.arange(N_TOKENS, dtype=jnp.int32) - start_unp[e_sorted])
    pos = jnp.zeros((N_TOKENS,), jnp.int32).at[order].set(pos_sorted)
    sorted_ids = jnp.zeros((N_BUF,), jnp.int32).at[pos].set(
        jnp.arange(N_TOKENS, dtype=jnp.int32))
    x_sorted = x[sorted_ids]
    w_sorted = jnp.zeros((N_BUF, 1), jnp.float32).at[pos].set(w)
    y_sorted = _expert_matmul(offsets, x_sorted, w_sorted, W, b)
    output = y_sorted[pos]
    return (output, logits)


# speedup vs baseline: 2.9543x; 1.1309x over previous
import functools

import jax
import jax.numpy as jnp
from jax import lax
from jax.experimental import pallas as pl
from jax.experimental.pallas import tpu as pltpu
from jax.experimental.pallas import tpu_sc as plsc

D_MODEL = 768
N_EXPERTS = 64
N_TOKENS = 2048
ROW_TILE = 64
N_BUF = 3072
PAD = 16
SENTINEL = N_TOKENS
N_CHUNKS = N_TOKENS // 16

_INTERPRET = False


def _router_body(x_ref, wr_ref, br_ref, logits_ref, w_ref, idx_ref, offs_ref):
    x = x_ref[...]
    logits = lax.dot_general(
        x, wr_ref[...], (((1,), (1,)), ((), ())),
        preferred_element_type=jnp.float32) + br_ref[...]
    logits_ref[...] = logits
    m = jnp.max(logits, axis=1, keepdims=True)
    p = jnp.exp(logits - m)
    s = jnp.sum(p, axis=1, keepdims=True)
    w_ref[...] = 1.0 / s
    lane = lax.broadcasted_iota(jnp.int32, (N_TOKENS, N_EXPERTS), 1)
    cand = jnp.where(logits == m, lane, N_EXPERTS)
    idx = jnp.min(cand, axis=1, keepdims=True)
    idx_ref[...] = idx
    onehot = (lane == idx).astype(jnp.float32)
    cnt = jnp.sum(onehot, axis=0, keepdims=True)
    padded = (cnt.astype(jnp.int32) + (PAD - 1)) & ~(PAD - 1)
    row_i = lax.broadcasted_iota(jnp.int32, (N_EXPERTS, 2 * N_EXPERTS), 0)
    col_i = lax.broadcasted_iota(jnp.int32, (N_EXPERTS, 2 * N_EXPERTS), 1)
    tri = (row_i < col_i).astype(jnp.float32)
    offs = lax.dot_general(
        padded.astype(jnp.float32), tri, (((1,), (0,)), ((), ())),
        preferred_element_type=jnp.float32)
    offs_ref[...] = offs.astype(jnp.int32)


def _router(x, Wr, br):
    return pl.pallas_call(
        _router_body,
        out_shape=[
            jax.ShapeDtypeStruct((N_TOKENS, N_EXPERTS), jnp.float32),
            jax.ShapeDtypeStruct((N_TOKENS, 1), jnp.float32),
            jax.ShapeDtypeStruct((N_TOKENS, 1), jnp.int32),
            jax.ShapeDtypeStruct((1, 2 * N_EXPERTS), jnp.int32),
        ],
        interpret=_INTERPRET,
    )(x, Wr, br.reshape(1, N_EXPERTS))


def _vext(ref, i):
    chunk = ref[pl.ds((i // 16) * 16, 16)]
    return jnp.sum(jnp.where(lax.iota(jnp.int32, 16) == i % 16, chunk, 0))


def _dispatch_body(idx_hbm, w_hbm, offs_hbm, x_hbm,
                   tok_hbm, ws_hbm, xs_hbm,
                   idx_v, w_v, offs_v, tok_v, wsl_v, gidx_v, rows_v, sem):
    wid = lax.axis_index("s") * 2 + lax.axis_index("c")
    pltpu.sync_copy(idx_hbm, idx_v)
    pltpu.sync_copy(w_hbm, w_v)
    pltpu.sync_copy(offs_hbm, offs_v)
    lanes = lax.iota(jnp.int32, 16)
    for eo in range(2):
        e = wid * 2 + eo
        off_e = _vext(offs_v, e)

        def scan_chunk(i, cnt, e=e):
            v = idx_v[pl.ds(i * 16, 16)]
            msk = v == e
            mi = jnp.where(msk, 1, 0)
            slots = cnt + plsc.cumsum(mi) - 1
            toks = lax.iota(jnp.int32, 16) + i * 16
            plsc.store_scatter(tok_v, [slots], toks, mask=msk)
            plsc.store_scatter(wsl_v, [slots], w_v[pl.ds(i * 16, 16)],
                               mask=msk)
            return cnt + jnp.sum(mi)

        cnt = lax.fori_loop(0, N_CHUNKS, scan_chunk, 0)
        padded = (cnt + (PAD - 1)) & ~(PAD - 1)
        gpos = (cnt // 16) * 16 + lanes
        m_pad = (gpos >= cnt) & (gpos < padded)
        plsc.store_scatter(tok_v, [gpos], jnp.full((16,), SENTINEL, jnp.int32),
                           mask=m_pad)
        plsc.store_scatter(wsl_v, [gpos], jnp.zeros((16,), jnp.float32),
                           mask=m_pad)

        def move_chunk(t, carry, off_e=off_e):
            base = t * 16
            dst = pl.multiple_of(off_e + base, 16)
            gidx_v[...] = jnp.minimum(tok_v[pl.ds(base, 16)], N_TOKENS - 1)
            pltpu.sync_copy(tok_v.at[pl.ds(base, 16)],
                            tok_hbm.at[pl.ds(dst, 16)])
            pltpu.sync_copy(wsl_v.at[pl.ds(base, 16)],
                            ws_hbm.at[pl.ds(dst, 16)])
            pltpu.async_copy(x_hbm.at[gidx_v], rows_v, sem).wait()
            pltpu.sync_copy(rows_v, xs_hbm.at[pl.ds(dst, 16)])
            return carry

        lax.fori_loop(0, padded // 16, move_chunk, 0)


def _dispatch(idx, w, offs, x):
    mesh = plsc.VectorSubcoreMesh(core_axis_name="c", subcore_axis_name="s")
    f = pl.kernel(
        _dispatch_body,
        out_type=[
            jax.ShapeDtypeStruct((N_BUF,), jnp.int32),
            jax.ShapeDtypeStruct((N_BUF,), jnp.float32),
            jax.ShapeDtypeStruct((N_BUF, D_MODEL), jnp.float32),
        ],
        mesh=mesh,
        compiler_params=pltpu.CompilerParams(needs_layout_passes=False),
        scratch_types=[
            pltpu.VMEM((N_TOKENS,), jnp.int32),
            pltpu.VMEM((N_TOKENS,), jnp.float32),
            pltpu.VMEM((2 * N_EXPERTS,), jnp.int32),
            pltpu.VMEM((N_TOKENS,), jnp.int32),
            pltpu.VMEM((N_TOKENS,), jnp.float32),
            pltpu.VMEM((16,), jnp.int32),
            pltpu.VMEM((16, D_MODEL), jnp.float32),
            pltpu.SemaphoreType.DMA,
        ],
    )
    return f(idx, w, offs, x)


def _expert_body(offs_ref, xs_ref, ws_ref, w_ref, b_ref, ys_ref):
    e = pl.program_id(0)
    start = offs_ref[e]
    stop = offs_ref[e + 1]
    n_tiles = (stop - start + ROW_TILE - 1) // ROW_TILE

    def tile(t, carry):
        s = pl.multiple_of(start + t * ROW_TILE, PAD)
        xt = xs_ref[pl.ds(s, ROW_TILE), :]
        y = lax.dot_general(
            xt, w_ref[0], (((1,), (1,)), ((), ())),
            preferred_element_type=jnp.float32)
        y = (y + b_ref[0]) * ws_ref[pl.ds(s, ROW_TILE), :]
        ys_ref[pl.ds(s, ROW_TILE), :] = y
        return carry

    lax.fori_loop(0, n_tiles, tile, 0)


def _expert_matmul(offsets, x_sorted, w_sorted, W, b):
    grid_spec = pltpu.PrefetchScalarGridSpec(
        num_scalar_prefetch=1,
        grid=(N_EXPERTS,),
        in_specs=[
            pl.BlockSpec((N_BUF, D_MODEL), lambda e, offs: (0, 0)),
            pl.BlockSpec((N_BUF, 1), lambda e, offs: (0, 0)),
            pl.BlockSpec((1, D_MODEL, D_MODEL), lambda e, offs: (e, 0, 0)),
            pl.BlockSpec((1, 1, D_MODEL), lambda e, offs: (e, 0, 0)),
        ],
        out_specs=pl.BlockSpec((N_BUF, D_MODEL), lambda e, offs: (0, 0)),
    )
    return pl.pallas_call(
        _expert_body,
        grid_spec=grid_spec,
        out_shape=jax.ShapeDtypeStruct((N_BUF, D_MODEL), jnp.float32),
        interpret=_INTERPRET,
    )(offsets, x_sorted, w_sorted, W, b.reshape(N_EXPERTS, 1, D_MODEL))


def _combine_body(ys_hbm, tok_hbm, offs_hbm, out_hbm,
                  offs_v, sidx_v, rows_v, sem):
    wid = lax.axis_index("s") * 2 + lax.axis_index("c")
    pltpu.sync_copy(offs_hbm, offs_v)
    for eo in range(2):
        e = wid * 2 + eo
        off_e = _vext(offs_v, e)
        n_ch = (_vext(offs_v, e + 1) - off_e) // 16

        def move_chunk(t, carry, off_e=off_e):
            src = pl.multiple_of(off_e + t * 16, 16)
            pltpu.sync_copy(tok_hbm.at[pl.ds(src, 16)], sidx_v)
            pltpu.sync_copy(ys_hbm.at[pl.ds(src, 16)], rows_v)
            pltpu.async_copy(rows_v, out_hbm.at[sidx_v], sem).wait()
            return carry

        lax.fori_loop(0, n_ch, move_chunk, 0)


def _combine(y_sorted, tok, offs):
    mesh = plsc.VectorSubcoreMesh(core_axis_name="c", subcore_axis_name="s")
    f = pl.kernel(
        _combine_body,
        out_type=jax.ShapeDtypeStruct((N_TOKENS + 16, D_MODEL), jnp.float32),
        mesh=mesh,
        compiler_params=pltpu.CompilerParams(needs_layout_passes=False),
        scratch_types=[
            pltpu.VMEM((2 * N_EXPERTS,), jnp.int32),
            pltpu.VMEM((16,), jnp.int32),
            pltpu.VMEM((16, D_MODEL), jnp.float32),
            pltpu.SemaphoreType.DMA,
        ],
    )
    return f(y_sorted, tok, offs)


def kernel(x, W, b, Wr, br):
    logits, w, idx, offs = _router(x, Wr, br)
    offs_flat = offs.reshape(2 * N_EXPERTS)
    tok, w_sorted, x_sorted = _dispatch(
        idx.reshape(N_TOKENS), w.reshape(N_TOKENS), offs_flat, x)
    y_sorted = _expert_matmul(
        offs_flat, x_sorted, w_sorted.reshape(N_BUF, 1), W, b)
    out_ext = _combine(y_sorted, tok, offs_flat)
    return (out_ext[:N_TOKENS], logits)


# speedup vs baseline: 3.0048x; 1.0171x over previous
import functools

import jax
import jax.numpy as jnp
from jax import lax
from jax.experimental import pallas as pl
from jax.experimental.pallas import tpu as pltpu
from jax.experimental.pallas import tpu_sc as plsc

D_MODEL = 768
N_EXPERTS = 64
N_TOKENS = 2048
ROW_TILE = 64
N_BUF = 3072
PAD = 16
SENTINEL = N_TOKENS
N_CHUNKS = N_TOKENS // 16

_INTERPRET = False


def _router_body(x_ref, wr_ref, br_ref, logits_ref, w_ref, idx_ref, offs_ref):
    x = x_ref[...]
    logits = lax.dot_general(
        x, wr_ref[...], (((1,), (1,)), ((), ())),
        preferred_element_type=jnp.float32) + br_ref[...]
    logits_ref[...] = logits
    m = jnp.max(logits, axis=1, keepdims=True)
    p = jnp.exp(logits - m)
    s = jnp.sum(p, axis=1, keepdims=True)
    w_ref[...] = 1.0 / s
    lane = lax.broadcasted_iota(jnp.int32, (N_TOKENS, N_EXPERTS), 1)
    cand = jnp.where(logits == m, lane, N_EXPERTS)
    idx = jnp.min(cand, axis=1, keepdims=True)
    idx_ref[...] = idx
    onehot = (lane == idx).astype(jnp.float32)
    cnt = jnp.sum(onehot, axis=0, keepdims=True)
    padded = (cnt.astype(jnp.int32) + (PAD - 1)) & ~(PAD - 1)
    row_i = lax.broadcasted_iota(jnp.int32, (N_EXPERTS, 2 * N_EXPERTS), 0)
    col_i = lax.broadcasted_iota(jnp.int32, (N_EXPERTS, 2 * N_EXPERTS), 1)
    tri = (row_i < col_i).astype(jnp.float32)
    offs = lax.dot_general(
        padded.astype(jnp.float32), tri, (((1,), (0,)), ((), ())),
        preferred_element_type=jnp.float32)
    offs_ref[...] = offs.astype(jnp.int32)


def _router(x, Wr, br):
    return pl.pallas_call(
        _router_body,
        out_shape=[
            jax.ShapeDtypeStruct((N_TOKENS, N_EXPERTS), jnp.float32),
            jax.ShapeDtypeStruct((N_TOKENS, 1), jnp.float32),
            jax.ShapeDtypeStruct((N_TOKENS, 1), jnp.int32),
            jax.ShapeDtypeStruct((1, 2 * N_EXPERTS), jnp.int32),
        ],
        interpret=_INTERPRET,
    )(x, Wr, br.reshape(1, N_EXPERTS))


def _vext(ref, i):
    chunk = ref[pl.ds((i // 16) * 16, 16)]
    return jnp.sum(jnp.where(lax.iota(jnp.int32, 16) == i % 16, chunk, 0))


K_GRP = 8


def _drain(n, src, dst, sem):
    def body(r, carry):
        pltpu.make_async_copy(src, dst, sem).wait()
        return carry

    lax.fori_loop(0, n, body, 0)


def _dispatch_body(idx_hbm, w_hbm, offs_hbm, x_hbm,
                   tok_hbm, ws_hbm, xs_hbm,
                   idx_v, w_v, offs_v, tok0_v, wsl0_v, tok1_v, wsl1_v,
                   rows_v, semg, sems, semm):
    wid = lax.axis_index("s") * 2 + lax.axis_index("c")
    pltpu.async_copy(idx_hbm, idx_v, semg)
    pltpu.async_copy(w_hbm, w_v, semg)
    pltpu.async_copy(offs_hbm, offs_v, semg)
    pltpu.make_async_copy(idx_hbm, idx_v, semg).wait()
    pltpu.make_async_copy(w_hbm, w_v, semg).wait()
    pltpu.make_async_copy(offs_hbm, offs_v, semg).wait()

    lanes = lax.iota(jnp.int32, 16)
    e0 = wid * 2

    def scan_chunk(i, c01):
        c0, c1 = c01
        v = idx_v[pl.ds(i * 16, 16)]
        wv = w_v[pl.ds(i * 16, 16)]
        toks = lanes + i * 16
        m0 = v == e0
        m1 = v == e0 + 1
        s0 = c0 + plsc.cumsum(jnp.where(m0, 1, 0)) - 1
        s1 = c1 + plsc.cumsum(jnp.where(m1, 1, 0)) - 1
        plsc.store_scatter(tok0_v, [s0], toks, mask=m0)
        plsc.store_scatter(wsl0_v, [s0], wv, mask=m0)
        plsc.store_scatter(tok1_v, [s1], toks, mask=m1)
        plsc.store_scatter(wsl1_v, [s1], wv, mask=m1)
        return (c0 + jnp.sum(jnp.where(m0, 1, 0)),
                c1 + jnp.sum(jnp.where(m1, 1, 0)))

    cnt0, cnt1 = lax.fori_loop(0, N_CHUNKS, scan_chunk, (0, 0))

    for cnt, off_e, tok_v, wsl_v in (
            (cnt0, _vext(offs_v, e0), tok0_v, wsl0_v),
            (cnt1, _vext(offs_v, e0 + 1), tok1_v, wsl1_v)):
        padded = (cnt + (PAD - 1)) & ~(PAD - 1)
        gpos = (cnt // 16) * 16 + lanes
        m_pad = (gpos >= cnt) & (gpos < padded)
        plsc.store_scatter(tok_v, [gpos],
                           jnp.full((16,), SENTINEL, jnp.int32), mask=m_pad)
        plsc.store_scatter(wsl_v, [gpos], jnp.zeros((16,), jnp.float32),
                           mask=m_pad)
        n_ch = padded // 16

        def group(g, carry, off_e=off_e, tok_v=tok_v, wsl_v=wsl_v):
            base_ch = g * K_GRP
            k_act = jnp.minimum(n_ch - base_ch, K_GRP)
            for j in range(K_GRP):
                @pl.when(j < k_act)
                def _():
                    b = (base_ch + j) * 16
                    dst = pl.multiple_of(off_e + b, 16)
                    pltpu.async_copy(tok_v.at[pl.ds(b, 16)],
                                     tok_hbm.at[pl.ds(dst, 16)], semm)
                    pltpu.async_copy(wsl_v.at[pl.ds(b, 16)],
                                     ws_hbm.at[pl.ds(dst, 16)], semm)
                    gidx = jnp.minimum(tok_v[pl.ds(b, 16)], N_TOKENS - 1)
                    pltpu.async_copy(x_hbm.at[gidx], rows_v.at[j], semg)
            _drain(k_act, x_hbm.at[pl.ds(0, 16)], rows_v.at[0], semg)
            for j in range(K_GRP):
                @pl.when(j < k_act)
                def _():
                    b = (base_ch + j) * 16
                    dst = pl.multiple_of(off_e + b, 16)
                    pltpu.async_copy(rows_v.at[j],
                                     xs_hbm.at[pl.ds(dst, 16)], sems)
            _drain(k_act, x_hbm.at[pl.ds(0, 16)], rows_v.at[0], sems)
            _drain(2 * k_act, tok_hbm.at[pl.ds(0, 16)],
                   idx_v.at[pl.ds(0, 16)], semm)
            return carry

        lax.fori_loop(0, (n_ch + K_GRP - 1) // K_GRP, group, 0)


def _dispatch(idx, w, offs, x):
    mesh = plsc.VectorSubcoreMesh(core_axis_name="c", subcore_axis_name="s")
    f = pl.kernel(
        _dispatch_body,
        out_type=[
            jax.ShapeDtypeStruct((N_BUF,), jnp.int32),
            jax.ShapeDtypeStruct((N_BUF,), jnp.float32),
            jax.ShapeDtypeStruct((N_BUF, D_MODEL), jnp.float32),
        ],
        mesh=mesh,
        compiler_params=pltpu.CompilerParams(needs_layout_passes=False),
        scratch_types=[
            pltpu.VMEM((N_TOKENS,), jnp.int32),
            pltpu.VMEM((N_TOKENS,), jnp.float32),
            pltpu.VMEM((2 * N_EXPERTS,), jnp.int32),
            pltpu.VMEM((N_TOKENS,), jnp.int32),
            pltpu.VMEM((N_TOKENS,), jnp.float32),
            pltpu.VMEM((N_TOKENS,), jnp.int32),
            pltpu.VMEM((N_TOKENS,), jnp.float32),
            pltpu.VMEM((K_GRP, 16, D_MODEL), jnp.float32),
            pltpu.SemaphoreType.DMA,
            pltpu.SemaphoreType.DMA,
            pltpu.SemaphoreType.DMA,
        ],
    )
    return f(idx, w, offs, x)


def _expert_body(offs_ref, xs_ref, ws_ref, w_ref, b_ref, ys_ref):
    e = pl.program_id(0)
    start = offs_ref[e]
    stop = offs_ref[e + 1]
    n_tiles = (stop - start + ROW_TILE - 1) // ROW_TILE

    def tile(t, carry):
        s = pl.multiple_of(start + t * ROW_TILE, PAD)
        xt = xs_ref[pl.ds(s, ROW_TILE), :]
        y = lax.dot_general(
            xt, w_ref[0], (((1,), (1,)), ((), ())),
            preferred_element_type=jnp.float32)
        y = (y + b_ref[0]) * ws_ref[pl.ds(s, ROW_TILE), :]
        ys_ref[pl.ds(s, ROW_TILE), :] = y
        return carry

    lax.fori_loop(0, n_tiles, tile, 0)


def _expert_matmul(offsets, x_sorted, w_sorted, W, b):
    grid_spec = pltpu.PrefetchScalarGridSpec(
        num_scalar_prefetch=1,
        grid=(N_EXPERTS,),
        in_specs=[
            pl.BlockSpec((N_BUF, D_MODEL), lambda e, offs: (0, 0)),
            pl.BlockSpec((N_BUF, 1), lambda e, offs: (0, 0)),
            pl.BlockSpec((1, D_MODEL, D_MODEL), lambda e, offs: (e, 0, 0)),
            pl.BlockSpec((1, 1, D_MODEL), lambda e, offs: (e, 0, 0)),
        ],
        out_specs=pl.BlockSpec((N_BUF, D_MODEL), lambda e, offs: (0, 0)),
    )
    return pl.pallas_call(
        _expert_body,
        grid_spec=grid_spec,
        out_shape=jax.ShapeDtypeStruct((N_BUF, D_MODEL), jnp.float32),
        interpret=_INTERPRET,
    )(offsets, x_sorted, w_sorted, W, b.reshape(N_EXPERTS, 1, D_MODEL))


def _combine_body(ys_hbm, tok_hbm, offs_hbm, out_hbm,
                  offs_v, sidx_v, rows_v, semg, sems):
    wid = lax.axis_index("s") * 2 + lax.axis_index("c")
    pltpu.sync_copy(offs_hbm, offs_v)
    for eo in range(2):
        e = wid * 2 + eo
        off_e = _vext(offs_v, e)
        n_ch = (_vext(offs_v, e + 1) - off_e) // 16

        def group(g, carry, off_e=off_e, n_ch=n_ch):
            base_ch = g * K_GRP
            k_act = jnp.minimum(n_ch - base_ch, K_GRP)
            for j in range(K_GRP):
                @pl.when(j < k_act)
                def _():
                    src = pl.multiple_of(off_e + (base_ch + j) * 16, 16)
                    pltpu.async_copy(tok_hbm.at[pl.ds(src, 16)],
                                     sidx_v.at[j], semg)
                    pltpu.async_copy(ys_hbm.at[pl.ds(src, 16)],
                                     rows_v.at[j], semg)
            _drain(k_act, ys_hbm.at[pl.ds(0, 16)], rows_v.at[0], semg)
            _drain(k_act, tok_hbm.at[pl.ds(0, 16)], sidx_v.at[0], semg)
            for j in range(K_GRP):
                @pl.when(j < k_act)
                def _():
                    sidx = sidx_v[j, :]
                    pltpu.async_copy(rows_v.at[j], out_hbm.at[sidx], sems)
            _drain(k_act, ys_hbm.at[pl.ds(0, 16)], rows_v.at[0], sems)
            return carry

        lax.fori_loop(0, (n_ch + K_GRP - 1) // K_GRP, group, 0)


def _combine(y_sorted, tok, offs):
    mesh = plsc.VectorSubcoreMesh(core_axis_name="c", subcore_axis_name="s")
    f = pl.kernel(
        _combine_body,
        out_type=jax.ShapeDtypeStruct((N_TOKENS + 16, D_MODEL), jnp.float32),
        mesh=mesh,
        compiler_params=pltpu.CompilerParams(needs_layout_passes=False),
        scratch_types=[
            pltpu.VMEM((2 * N_EXPERTS,), jnp.int32),
            pltpu.VMEM((K_GRP, 16), jnp.int32),
            pltpu.VMEM((K_GRP, 16, D_MODEL), jnp.float32),
            pltpu.SemaphoreType.DMA,
            pltpu.SemaphoreType.DMA,
        ],
    )
    return f(y_sorted, tok, offs)


def kernel(x, W, b, Wr, br):
    logits, w, idx, offs = _router(x, Wr, br)
    offs_flat = offs.reshape(2 * N_EXPERTS)
    tok, w_sorted, x_sorted = _dispatch(
        idx.reshape(N_TOKENS), w.reshape(N_TOKENS), offs_flat, x)
    y_sorted = _expert_matmul(
        offs_flat, x_sorted, w_sorted.reshape(N_BUF, 1), W, b)
    out_ext = _combine(y_sorted, tok, offs_flat)
    return (out_ext[:N_TOKENS], logits)


# speedup vs baseline: 4.5710x; 1.5212x over previous
import jax
import jax.numpy as jnp
from jax import lax
from jax.experimental import pallas as pl
from jax.experimental.pallas import tpu as pltpu
from jax.experimental.pallas import tpu_sc as plsc

D_MODEL = 768
N_EXPERTS = 64
N_TOKENS = 2048
ROW_TILE = 64
N_BUF = 3072
PAD = 16
TOK_PER_W = 64
RBLK = 128

_INTERPRET = False


def _router_body(x_ref, wr_ref, br_ref, logits_ref, w_ref, pos_ref, offs_ref):
    x = x_ref[...]
    logits = lax.dot_general(
        x, wr_ref[...], (((1,), (1,)), ((), ())),
        preferred_element_type=jnp.float32) + br_ref[...]
    logits_ref[...] = logits
    m = jnp.max(logits, axis=1, keepdims=True)
    p = jnp.exp(logits - m)
    s = jnp.sum(p, axis=1, keepdims=True)
    w_ref[...] = 1.0 / s
    lane = lax.broadcasted_iota(jnp.int32, (N_TOKENS, N_EXPERTS), 1)
    cand = jnp.where(logits == m, lane, N_EXPERTS)
    idx = jnp.min(cand, axis=1, keepdims=True)
    onehot = (lane == idx).astype(jnp.float32)
    cnt = jnp.sum(onehot, axis=0, keepdims=True)
    padded = ((cnt.astype(jnp.int32) + (PAD - 1)) & ~(PAD - 1)).astype(
        jnp.float32)
    row_i = lax.broadcasted_iota(jnp.int32, (N_EXPERTS, 2 * N_EXPERTS), 0)
    col_i = lax.broadcasted_iota(jnp.int32, (N_EXPERTS, 2 * N_EXPERTS), 1)
    tri = (row_i < col_i).astype(jnp.float32)
    offs = lax.dot_general(
        padded, tri, (((1,), (0,)), ((), ())),
        preferred_element_type=jnp.float32)
    offs_ref[...] = offs.astype(jnp.int32)
    ri = lax.broadcasted_iota(jnp.int32, (RBLK, RBLK), 0)
    ci = lax.broadcasted_iota(jnp.int32, (RBLK, RBLK), 1)
    tri_b = (ci < ri).astype(jnp.float32)
    offs64 = offs[:, :N_EXPERTS]
    base = jnp.zeros((1, N_EXPERTS), jnp.float32)
    for t in range(N_TOKENS // RBLK):
        oh_t = onehot[t * RBLK:(t + 1) * RBLK, :]
        within = lax.dot_general(
            tri_b, oh_t, (((1,), (0,)), ((), ())),
            preferred_element_type=jnp.float32)
        pos_t = jnp.sum(oh_t * (within + base + offs64), axis=1,
                        keepdims=True)
        pos_ref[t * RBLK:(t + 1) * RBLK, :] = pos_t.astype(jnp.int32)
        base = base + jnp.sum(oh_t, axis=0, keepdims=True)


def _router(x, Wr, br):
    return pl.pallas_call(
        _router_body,
        out_shape=[
            jax.ShapeDtypeStruct((N_TOKENS, N_EXPERTS), jnp.float32),
            jax.ShapeDtypeStruct((N_TOKENS, 1), jnp.float32),
            jax.ShapeDtypeStruct((N_TOKENS, 1), jnp.int32),
            jax.ShapeDtypeStruct((1, 2 * N_EXPERTS), jnp.int32),
        ],
        interpret=_INTERPRET,
    )(x, Wr, br.reshape(1, N_EXPERTS))


def _dispatch_body(pos_hbm, w_hbm, x_hbm, ws_hbm, xs_hbm,
                   pos_v, w_v, xrows_v, wbuf_v, semg, sems):
    wid = lax.axis_index("s") * 2 + lax.axis_index("c")
    base = pl.multiple_of(wid * TOK_PER_W, TOK_PER_W)
    pltpu.async_copy(pos_hbm.at[pl.ds(base, TOK_PER_W)], pos_v, semg)
    pltpu.async_copy(w_hbm.at[pl.ds(base, TOK_PER_W)], w_v, semg)
    pltpu.async_copy(x_hbm.at[pl.ds(base, TOK_PER_W)], xrows_v, semg)
    pltpu.make_async_copy(pos_hbm.at[pl.ds(base, TOK_PER_W)], pos_v,
                          semg).wait()
    pltpu.make_async_copy(w_hbm.at[pl.ds(base, TOK_PER_W)], w_v, semg).wait()
    pltpu.make_async_copy(x_hbm.at[pl.ds(base, TOK_PER_W)], xrows_v,
                          semg).wait()
    lanes = lax.iota(jnp.int32, 16)
    zeros = jnp.zeros((16,), jnp.int32)
    for j in range(TOK_PER_W // 16):
        plsc.store_scatter(wbuf_v, [lanes + j * 16, zeros],
                           w_v[pl.ds(j * 16, 16)])
    for j in range(TOK_PER_W // 16):
        posvec = pos_v[pl.ds(j * 16, 16)]
        pltpu.async_copy(xrows_v.at[pl.ds(j * 16, 16)],
                         xs_hbm.at[posvec], sems)
        pltpu.async_copy(wbuf_v.at[pl.ds(j * 16, 16)],
                         ws_hbm.at[posvec], sems)
    for j in range(TOK_PER_W // 16):
        pltpu.make_async_copy(xs_hbm.at[pl.ds(0, 16)],
                              xrows_v.at[pl.ds(0, 16)], sems).wait()
        pltpu.make_async_copy(ws_hbm.at[pl.ds(0, 16)],
                              wbuf_v.at[pl.ds(0, 16)], sems).wait()


def _dispatch(pos, w, x):
    mesh = plsc.VectorSubcoreMesh(core_axis_name="c", subcore_axis_name="s")
    f = pl.kernel(
        _dispatch_body,
        out_type=[
            jax.ShapeDtypeStruct((N_BUF, 128), jnp.float32),
            jax.ShapeDtypeStruct((N_BUF, D_MODEL), jnp.float32),
        ],
        mesh=mesh,
        compiler_params=pltpu.CompilerParams(needs_layout_passes=False),
        scratch_types=[
            pltpu.VMEM((TOK_PER_W,), jnp.int32),
            pltpu.VMEM((TOK_PER_W,), jnp.float32),
            pltpu.VMEM((TOK_PER_W, D_MODEL), jnp.float32),
            pltpu.VMEM((TOK_PER_W, 128), jnp.float32),
            pltpu.SemaphoreType.DMA,
            pltpu.SemaphoreType.DMA,
        ],
    )
    return f(pos, w, x)


def _expert_body(offs_ref, xs_ref, ws_ref, w_ref, b_ref, ys_ref):
    e = pl.program_id(0)
    start = offs_ref[e]
    stop = offs_ref[e + 1]
    n_tiles = (stop - start + ROW_TILE - 1) // ROW_TILE

    def tile(t, carry):
        s = pl.multiple_of(start + t * ROW_TILE, PAD)
        xt = xs_ref[pl.ds(s, ROW_TILE), :]
        y = lax.dot_general(
            xt, w_ref[0], (((1,), (1,)), ((), ())),
            preferred_element_type=jnp.float32)
        y = (y + b_ref[0]) * ws_ref[pl.ds(s, ROW_TILE), 0:1]
        ys_ref[pl.ds(s, ROW_TILE), :] = y
        return carry

    lax.fori_loop(0, n_tiles, tile, 0)


def _expert_matmul(offsets, x_sorted, w_sorted, W, b):
    grid_spec = pltpu.PrefetchScalarGridSpec(
        num_scalar_prefetch=1,
        grid=(N_EXPERTS,),
        in_specs=[
            pl.BlockSpec((N_BUF, D_MODEL), lambda e, offs: (0, 0)),
            pl.BlockSpec((N_BUF, 128), lambda e, offs: (0, 0)),
            pl.BlockSpec((1, D_MODEL, D_MODEL), lambda e, offs: (e, 0, 0)),
            pl.BlockSpec((1, 1, D_MODEL), lambda e, offs: (e, 0, 0)),
        ],
        out_specs=pl.BlockSpec((N_BUF, D_MODEL), lambda e, offs: (0, 0)),
    )
    return pl.pallas_call(
        _expert_body,
        grid_spec=grid_spec,
        out_shape=jax.ShapeDtypeStruct((N_BUF, D_MODEL), jnp.float32),
        interpret=_INTERPRET,
    )(offsets, x_sorted, w_sorted, W, b.reshape(N_EXPERTS, 1, D_MODEL))


def _combine_body(ys_hbm, pos_hbm, out_hbm, pos_v, yrows_v, semg):
    wid = lax.axis_index("s") * 2 + lax.axis_index("c")
    base = pl.multiple_of(wid * TOK_PER_W, TOK_PER_W)
    pltpu.async_copy(pos_hbm.at[pl.ds(base, TOK_PER_W)], pos_v, semg)
    pltpu.make_async_copy(pos_hbm.at[pl.ds(base, TOK_PER_W)], pos_v,
                          semg).wait()
    for j in range(TOK_PER_W // 16):
        posvec = pos_v[pl.ds(j * 16, 16)]
        pltpu.async_copy(ys_hbm.at[posvec],
                         yrows_v.at[pl.ds(j * 16, 16)], semg)
    for j in range(TOK_PER_W // 16):
        pltpu.make_async_copy(ys_hbm.at[pl.ds(0, 16)],
                              yrows_v.at[pl.ds(0, 16)], semg).wait()
    pltpu.sync_copy(yrows_v, out_hbm.at[pl.ds(base, TOK_PER_W)])


def _combine(y_sorted, pos):
    mesh = plsc.VectorSubcoreMesh(core_axis_name="c", subcore_axis_name="s")
    f = pl.kernel(
        _combine_body,
        out_type=jax.ShapeDtypeStruct((N_TOKENS, D_MODEL), jnp.float32),
        mesh=mesh,
        compiler_params=pltpu.CompilerParams(needs_layout_passes=False),
        scratch_types=[
            pltpu.VMEM((TOK_PER_W,), jnp.int32),
            pltpu.VMEM((TOK_PER_W, D_MODEL), jnp.float32),
            pltpu.SemaphoreType.DMA,
        ],
    )
    return f(y_sorted, pos)


def kernel(x, W, b, Wr, br):
    logits, w, pos, offs = _router(x, Wr, br)
    pos_flat = pos.reshape(N_TOKENS)
    offs_flat = offs.reshape(2 * N_EXPERTS)
    w_sorted, x_sorted = _dispatch(pos_flat, w.reshape(N_TOKENS), x)
    y_sorted = _expert_matmul(offs_flat, x_sorted, w_sorted, W, b)
    output = _combine(y_sorted, pos_flat)
    return (output, logits)


# speedup vs baseline: 4.6013x; 1.0066x over previous
import jax
import jax.numpy as jnp
from jax import lax
from jax.experimental import pallas as pl
from jax.experimental.pallas import tpu as pltpu
from jax.experimental.pallas import tpu_sc as plsc

D_MODEL = 768
N_EXPERTS = 64
N_TOKENS = 2048
ROW_TILE = 64
N_BUF = 2560
PAD = 8
TOK_PER_W = 64
RBLK = 128

_INTERPRET = False


def _router_body(x_ref, wr_ref, br_ref, logits_ref, w_ref, pos_ref, offs_ref):
    x = x_ref[...]
    logits = lax.dot_general(
        x, wr_ref[...], (((1,), (1,)), ((), ())),
        preferred_element_type=jnp.float32) + br_ref[...]
    logits_ref[...] = logits
    m = jnp.max(logits, axis=1, keepdims=True)
    p = jnp.exp(logits - m)
    s = jnp.sum(p, axis=1, keepdims=True)
    w_ref[...] = 1.0 / s
    lane = lax.broadcasted_iota(jnp.int32, (N_TOKENS, N_EXPERTS), 1)
    cand = jnp.where(logits == m, lane, N_EXPERTS)
    idx = jnp.min(cand, axis=1, keepdims=True)
    onehot = (lane == idx).astype(jnp.float32)
    cnt = jnp.sum(onehot, axis=0, keepdims=True)
    padded = ((cnt.astype(jnp.int32) + (PAD - 1)) & ~(PAD - 1)).astype(
        jnp.float32)
    row_i = lax.broadcasted_iota(jnp.int32, (N_EXPERTS, 2 * N_EXPERTS), 0)
    col_i = lax.broadcasted_iota(jnp.int32, (N_EXPERTS, 2 * N_EXPERTS), 1)
    tri = (row_i < col_i).astype(jnp.float32)
    offs = lax.dot_general(
        padded, tri, (((1,), (0,)), ((), ())),
        preferred_element_type=jnp.float32)
    offs_ref[...] = offs.astype(jnp.int32)
    ri = lax.broadcasted_iota(jnp.int32, (RBLK, RBLK), 0)
    ci = lax.broadcasted_iota(jnp.int32, (RBLK, RBLK), 1)
    tri_b = (ci < ri).astype(jnp.float32)
    offs64 = offs[:, :N_EXPERTS]
    base = jnp.zeros((1, N_EXPERTS), jnp.float32)
    for t in range(N_TOKENS // RBLK):
        oh_t = onehot[t * RBLK:(t + 1) * RBLK, :]
        within = lax.dot_general(
            tri_b, oh_t, (((1,), (0,)), ((), ())),
            preferred_element_type=jnp.float32)
        pos_t = jnp.sum(oh_t * (within + base + offs64), axis=1,
                        keepdims=True)
        pos_ref[t * RBLK:(t + 1) * RBLK, :] = pos_t.astype(jnp.int32)
        base = base + jnp.sum(oh_t, axis=0, keepdims=True)


def _router(x, Wr, br):
    return pl.pallas_call(
        _router_body,
        out_shape=[
            jax.ShapeDtypeStruct((N_TOKENS, N_EXPERTS), jnp.float32),
            jax.ShapeDtypeStruct((N_TOKENS, 1), jnp.float32),
            jax.ShapeDtypeStruct((N_TOKENS, 1), jnp.int32),
            jax.ShapeDtypeStruct((1, 2 * N_EXPERTS), jnp.int32),
        ],
        interpret=_INTERPRET,
    )(x, Wr, br.reshape(1, N_EXPERTS))


def _dispatch_body(pos_hbm, w_hbm, x_hbm, ws_hbm, xs_hbm,
                   pos_v, w_v, xrows_v, wbuf_v, semg, sems):
    wid = lax.axis_index("s") * 2 + lax.axis_index("c")
    base = pl.multiple_of(wid * TOK_PER_W, TOK_PER_W)
    pltpu.async_copy(pos_hbm.at[pl.ds(base, TOK_PER_W)], pos_v, semg)
    pltpu.async_copy(w_hbm.at[pl.ds(base, TOK_PER_W)], w_v, semg)
    pltpu.async_copy(x_hbm.at[pl.ds(base, TOK_PER_W)], xrows_v, semg)
    pltpu.make_async_copy(pos_hbm.at[pl.ds(base, TOK_PER_W)], pos_v,
                          semg).wait()
    pltpu.make_async_copy(w_hbm.at[pl.ds(base, TOK_PER_W)], w_v, semg).wait()
    pltpu.make_async_copy(x_hbm.at[pl.ds(base, TOK_PER_W)], xrows_v,
                          semg).wait()
    lanes = lax.iota(jnp.int32, 16)
    zeros = jnp.zeros((16,), jnp.int32)
    for j in range(TOK_PER_W // 16):
        plsc.store_scatter(wbuf_v, [lanes + j * 16, zeros],
                           w_v[pl.ds(j * 16, 16)])
    for j in range(TOK_PER_W // 16):
        posvec = pos_v[pl.ds(j * 16, 16)]
        pltpu.async_copy(xrows_v.at[pl.ds(j * 16, 16)],
                         xs_hbm.at[posvec], sems)
        pltpu.async_copy(wbuf_v.at[pl.ds(j * 16, 16)],
                         ws_hbm.at[posvec], sems)
    for j in range(TOK_PER_W // 16):
        pltpu.make_async_copy(xs_hbm.at[pl.ds(0, 16)],
                              xrows_v.at[pl.ds(0, 16)], sems).wait()
        pltpu.make_async_copy(ws_hbm.at[pl.ds(0, 16)],
                              wbuf_v.at[pl.ds(0, 16)], sems).wait()


def _dispatch(pos, w, x):
    mesh = plsc.VectorSubcoreMesh(core_axis_name="c", subcore_axis_name="s")
    f = pl.kernel(
        _dispatch_body,
        out_type=[
            jax.ShapeDtypeStruct((N_BUF, 128), jnp.float32),
            jax.ShapeDtypeStruct((N_BUF, D_MODEL), jnp.float32),
        ],
        mesh=mesh,
        compiler_params=pltpu.CompilerParams(needs_layout_passes=False),
        scratch_types=[
            pltpu.VMEM((TOK_PER_W,), jnp.int32),
            pltpu.VMEM((TOK_PER_W,), jnp.float32),
            pltpu.VMEM((TOK_PER_W, D_MODEL), jnp.float32),
            pltpu.VMEM((TOK_PER_W, 128), jnp.float32),
            pltpu.SemaphoreType.DMA,
            pltpu.SemaphoreType.DMA,
        ],
    )
    return f(pos, w, x)


def _expert_body(offs_ref, xs_ref, ws_ref, w_ref, b_ref, ys_ref):
    e = pl.program_id(0)
    start = offs_ref[e]
    stop = offs_ref[e + 1]
    n_tiles = (stop - start + ROW_TILE - 1) // ROW_TILE

    def tile(t, carry):
        s = pl.multiple_of(start + t * ROW_TILE, PAD)
        xt = xs_ref[pl.ds(s, ROW_TILE), :]
        y = lax.dot_general(
            xt, w_ref[0], (((1,), (1,)), ((), ())),
            preferred_element_type=jnp.float32)
        y = (y + b_ref[0]) * ws_ref[pl.ds(s, ROW_TILE), 0:1]
        ys_ref[pl.ds(s, ROW_TILE), :] = y
        return carry

    lax.fori_loop(0, n_tiles, tile, 0)


def _expert_matmul(offsets, x_sorted, w_sorted, W, b):
    grid_spec = pltpu.PrefetchScalarGridSpec(
        num_scalar_prefetch=1,
        grid=(N_EXPERTS,),
        in_specs=[
            pl.BlockSpec((N_BUF, D_MODEL), lambda e, offs: (0, 0)),
            pl.BlockSpec((N_BUF, 128), lambda e, offs: (0, 0)),
            pl.BlockSpec((1, D_MODEL, D_MODEL), lambda e, offs: (e, 0, 0)),
            pl.BlockSpec((1, 1, D_MODEL), lambda e, offs: (e, 0, 0)),
        ],
        out_specs=pl.BlockSpec((N_BUF, D_MODEL), lambda e, offs: (0, 0)),
    )
    return pl.pallas_call(
        _expert_body,
        grid_spec=grid_spec,
        out_shape=jax.ShapeDtypeStruct((N_BUF, D_MODEL), jnp.float32),
        interpret=_INTERPRET,
    )(offsets, x_sorted, w_sorted, W, b.reshape(N_EXPERTS, 1, D_MODEL))


def _combine_body(ys_hbm, pos_hbm, out_hbm, pos_v, yrows_v, semg):
    wid = lax.axis_index("s") * 2 + lax.axis_index("c")
    base = pl.multiple_of(wid * TOK_PER_W, TOK_PER_W)
    pltpu.async_copy(pos_hbm.at[pl.ds(base, TOK_PER_W)], pos_v, semg)
    pltpu.make_async_copy(pos_hbm.at[pl.ds(base, TOK_PER_W)], pos_v,
                          semg).wait()
    for j in range(TOK_PER_W // 16):
        posvec = pos_v[pl.ds(j * 16, 16)]
        pltpu.async_copy(ys_hbm.at[posvec],
                         yrows_v.at[pl.ds(j * 16, 16)], semg)
    for j in range(TOK_PER_W // 16):
        pltpu.make_async_copy(ys_hbm.at[pl.ds(0, 16)],
                              yrows_v.at[pl.ds(0, 16)], semg).wait()
    pltpu.sync_copy(yrows_v, out_hbm.at[pl.ds(base, TOK_PER_W)])


def _combine(y_sorted, pos):
    mesh = plsc.VectorSubcoreMesh(core_axis_name="c", subcore_axis_name="s")
    f = pl.kernel(
        _combine_body,
        out_type=jax.ShapeDtypeStruct((N_TOKENS, D_MODEL), jnp.float32),
        mesh=mesh,
        compiler_params=pltpu.CompilerParams(needs_layout_passes=False),
        scratch_types=[
            pltpu.VMEM((TOK_PER_W,), jnp.int32),
            pltpu.VMEM((TOK_PER_W, D_MODEL), jnp.float32),
            pltpu.SemaphoreType.DMA,
        ],
    )
    return f(y_sorted, pos)


def kernel(x, W, b, Wr, br):
    logits, w, pos, offs = _router(x, Wr, br)
    pos_flat = pos.reshape(N_TOKENS)
    offs_flat = offs.reshape(2 * N_EXPERTS)
    w_sorted, x_sorted = _dispatch(pos_flat, w.reshape(N_TOKENS), x)
    y_sorted = _expert_matmul(offs_flat, x_sorted, w_sorted, W, b)
    output = _combine(y_sorted, pos_flat)
    return (output, logits)


# speedup vs baseline: 4.6277x; 1.0057x over previous
import jax
import jax.numpy as jnp
from jax import lax
from jax.experimental import pallas as pl
from jax.experimental.pallas import tpu as pltpu
from jax.experimental.pallas import tpu_sc as plsc

D_MODEL = 768
N_EXPERTS = 64
N_TOKENS = 2048
ROW_TILE = 64
N_BUF = 2560
PAD = 8
TOK_PER_W = 64
RBLK = 128

_INTERPRET = False


def _router_body(x_ref, wr_ref, br_ref, logits_ref, w_ref, pos_ref, offs_ref):
    x = x_ref[...]
    logits = lax.dot_general(
        x, wr_ref[...], (((1,), (1,)), ((), ())),
        preferred_element_type=jnp.float32) + br_ref[...]
    logits_ref[...] = logits
    m = jnp.max(logits, axis=1, keepdims=True)
    p = jnp.exp(logits - m)
    s = jnp.sum(p, axis=1, keepdims=True)
    w_ref[...] = 1.0 / s
    lane = lax.broadcasted_iota(jnp.int32, (N_TOKENS, N_EXPERTS), 1)
    cand = jnp.where(logits == m, lane, N_EXPERTS)
    idx = jnp.min(cand, axis=1, keepdims=True)
    onehot = (lane == idx).astype(jnp.float32)
    cnt = jnp.sum(onehot, axis=0, keepdims=True)
    padded = ((cnt.astype(jnp.int32) + (PAD - 1)) & ~(PAD - 1)).astype(
        jnp.float32)
    row_i = lax.broadcasted_iota(jnp.int32, (N_EXPERTS, 2 * N_EXPERTS), 0)
    col_i = lax.broadcasted_iota(jnp.int32, (N_EXPERTS, 2 * N_EXPERTS), 1)
    tri = (row_i < col_i).astype(jnp.float32)
    offs = lax.dot_general(
        padded, tri, (((1,), (0,)), ((), ())),
        preferred_element_type=jnp.float32)
    offs_ref[...] = offs.astype(jnp.int32)
    ri = lax.broadcasted_iota(jnp.int32, (RBLK, RBLK), 0)
    ci = lax.broadcasted_iota(jnp.int32, (RBLK, RBLK), 1)
    tri_b = (ci < ri).astype(jnp.float32)
    offs64 = offs[:, :N_EXPERTS]
    base = jnp.zeros((1, N_EXPERTS), jnp.float32)
    for t in range(N_TOKENS // RBLK):
        oh_t = onehot[t * RBLK:(t + 1) * RBLK, :]
        within = lax.dot_general(
            tri_b, oh_t, (((1,), (0,)), ((), ())),
            preferred_element_type=jnp.float32)
        pos_t = jnp.sum(oh_t * (within + base + offs64), axis=1,
                        keepdims=True)
        pos_ref[t * RBLK:(t + 1) * RBLK, :] = pos_t.astype(jnp.int32)
        base = base + jnp.sum(oh_t, axis=0, keepdims=True)


def _router(x, Wr, br):
    return pl.pallas_call(
        _router_body,
        out_shape=[
            jax.ShapeDtypeStruct((N_TOKENS, N_EXPERTS), jnp.float32),
            jax.ShapeDtypeStruct((N_TOKENS, 1), jnp.float32),
            jax.ShapeDtypeStruct((N_TOKENS, 1), jnp.int32),
            jax.ShapeDtypeStruct((1, 2 * N_EXPERTS), jnp.int32),
        ],
        interpret=_INTERPRET,
    )(x, Wr, br.reshape(1, N_EXPERTS))


def _dispatch_body(pos_hbm, w_hbm, x_hbm, ws_hbm, xs_hbm,
                   pos_v, w_v, xrows_v, wbuf_v, semg, sems):
    wid = lax.axis_index("s") * 2 + lax.axis_index("c")
    base = pl.multiple_of(wid * TOK_PER_W, TOK_PER_W)
    pltpu.async_copy(pos_hbm.at[pl.ds(base, TOK_PER_W)], pos_v, semg)
    pltpu.async_copy(w_hbm.at[pl.ds(base, TOK_PER_W)], w_v, semg)
    pltpu.async_copy(x_hbm.at[pl.ds(base, TOK_PER_W)], xrows_v, semg)
    pltpu.make_async_copy(pos_hbm.at[pl.ds(base, TOK_PER_W)], pos_v,
                          semg).wait()
    pltpu.make_async_copy(w_hbm.at[pl.ds(base, TOK_PER_W)], w_v, semg).wait()
    pltpu.make_async_copy(x_hbm.at[pl.ds(base, TOK_PER_W)], xrows_v,
                          semg).wait()
    lanes = lax.iota(jnp.int32, 16)
    zeros = jnp.zeros((16,), jnp.int32)
    for j in range(TOK_PER_W // 16):
        plsc.store_scatter(wbuf_v, [lanes + j * 16, zeros],
                           w_v[pl.ds(j * 16, 16)])
    for j in range(TOK_PER_W // 16):
        posvec = pos_v[pl.ds(j * 16, 16)]
        pltpu.async_copy(xrows_v.at[pl.ds(j * 16, 16)],
                         xs_hbm.at[posvec], sems)
        pltpu.async_copy(wbuf_v.at[pl.ds(j * 16, 16)],
                         ws_hbm.at[posvec], sems)
    for j in range(TOK_PER_W // 16):
        pltpu.make_async_copy(xs_hbm.at[pl.ds(0, 16)],
                              xrows_v.at[pl.ds(0, 16)], sems).wait()
        pltpu.make_async_copy(ws_hbm.at[pl.ds(0, 16)],
                              wbuf_v.at[pl.ds(0, 16)], sems).wait()


def _dispatch(pos, w, x):
    mesh = plsc.VectorSubcoreMesh(core_axis_name="c", subcore_axis_name="s")
    f = pl.kernel(
        _dispatch_body,
        out_type=[
            jax.ShapeDtypeStruct((N_BUF, 128), jnp.float32),
            jax.ShapeDtypeStruct((N_BUF, D_MODEL), jnp.float32),
        ],
        mesh=mesh,
        compiler_params=pltpu.CompilerParams(needs_layout_passes=False),
        scratch_types=[
            pltpu.VMEM((TOK_PER_W,), jnp.int32),
            pltpu.VMEM((TOK_PER_W,), jnp.float32),
            pltpu.VMEM((TOK_PER_W, D_MODEL), jnp.float32),
            pltpu.VMEM((TOK_PER_W, 128), jnp.float32),
            pltpu.SemaphoreType.DMA,
            pltpu.SemaphoreType.DMA,
        ],
    )
    return f(pos, w, x)


XCH = N_BUF // 8


def _expert_body(offs_ref, xs_hbm, ws_ref, w_ref, b_ref, ys_ref,
                 xbuf, semx, dr_ref):
    e = pl.program_id(0)
    start = offs_ref[e]
    stop = offs_ref[e + 1]

    @pl.when(e == 0)
    def _():
        dr_ref[0] = 0
        for j in range(8):
            pltpu.make_async_copy(
                xs_hbm.at[pl.ds(j * XCH, XCH)],
                xbuf.at[pl.ds(j * XCH, XCH)], semx).start()

    need = jnp.where(e == N_EXPERTS - 1, 8,
                     jnp.minimum((stop + ROW_TILE - PAD + XCH - 1) // XCH, 8))
    done = dr_ref[0]

    def drain(r, carry):
        pltpu.make_async_copy(xs_hbm.at[pl.ds(0, XCH)],
                              xbuf.at[pl.ds(0, XCH)], semx).wait()
        return carry

    lax.fori_loop(0, need - done, drain, 0)
    dr_ref[0] = jnp.maximum(need, done)

    n_tiles = (stop - start + ROW_TILE - 1) // ROW_TILE

    def tile(t, carry):
        s = pl.multiple_of(start + t * ROW_TILE, PAD)
        xt = xbuf[pl.ds(s, ROW_TILE), :]
        y = lax.dot_general(
            xt, w_ref[0], (((1,), (1,)), ((), ())),
            preferred_element_type=jnp.float32)
        y = (y + b_ref[0]) * ws_ref[pl.ds(s, ROW_TILE), 0:1]
        ys_ref[pl.ds(s, ROW_TILE), :] = y
        return carry

    lax.fori_loop(0, n_tiles, tile, 0)


def _expert_matmul(offsets, x_sorted, w_sorted, W, b):
    grid_spec = pltpu.PrefetchScalarGridSpec(
        num_scalar_prefetch=1,
        grid=(N_EXPERTS,),
        in_specs=[
            pl.BlockSpec(memory_space=pl.ANY),
            pl.BlockSpec((N_BUF, 128), lambda e, offs: (0, 0)),
            pl.BlockSpec((1, D_MODEL, D_MODEL), lambda e, offs: (e, 0, 0)),
            pl.BlockSpec((1, 1, D_MODEL), lambda e, offs: (e, 0, 0)),
        ],
        out_specs=pl.BlockSpec((N_BUF, D_MODEL), lambda e, offs: (0, 0)),
        scratch_shapes=[
            pltpu.VMEM((N_BUF, D_MODEL), jnp.float32),
            pltpu.SemaphoreType.DMA,
            pltpu.SMEM((1,), jnp.int32),
        ],
    )
    return pl.pallas_call(
        _expert_body,
        grid_spec=grid_spec,
        out_shape=jax.ShapeDtypeStruct((N_BUF, D_MODEL), jnp.float32),
        interpret=_INTERPRET,
    )(offsets, x_sorted, w_sorted, W, b.reshape(N_EXPERTS, 1, D_MODEL))


def _combine_body(ys_hbm, pos_hbm, out_hbm, pos_v, yrows_v, semg):
    wid = lax.axis_index("s") * 2 + lax.axis_index("c")
    base = pl.multiple_of(wid * TOK_PER_W, TOK_PER_W)
    pltpu.async_copy(pos_hbm.at[pl.ds(base, TOK_PER_W)], pos_v, semg)
    pltpu.make_async_copy(pos_hbm.at[pl.ds(base, TOK_PER_W)], pos_v,
                          semg).wait()
    for j in range(TOK_PER_W // 16):
        posvec = pos_v[pl.ds(j * 16, 16)]
        pltpu.async_copy(ys_hbm.at[posvec],
                         yrows_v.at[pl.ds(j * 16, 16)], semg)
    for j in range(TOK_PER_W // 16):
        pltpu.make_async_copy(ys_hbm.at[pl.ds(0, 16)],
                              yrows_v.at[pl.ds(0, 16)], semg).wait()
    pltpu.sync_copy(yrows_v, out_hbm.at[pl.ds(base, TOK_PER_W)])


def _combine(y_sorted, pos):
    mesh = plsc.VectorSubcoreMesh(core_axis_name="c", subcore_axis_name="s")
    f = pl.kernel(
        _combine_body,
        out_type=jax.ShapeDtypeStruct((N_TOKENS, D_MODEL), jnp.float32),
        mesh=mesh,
        compiler_params=pltpu.CompilerParams(needs_layout_passes=False),
        scratch_types=[
            pltpu.VMEM((TOK_PER_W,), jnp.int32),
            pltpu.VMEM((TOK_PER_W, D_MODEL), jnp.float32),
            pltpu.SemaphoreType.DMA,
        ],
    )
    return f(y_sorted, pos)


def kernel(x, W, b, Wr, br):
    logits, w, pos, offs = _router(x, Wr, br)
    pos_flat = pos.reshape(N_TOKENS)
    offs_flat = offs.reshape(2 * N_EXPERTS)
    w_sorted, x_sorted = _dispatch(pos_flat, w.reshape(N_TOKENS), x)
    y_sorted = _expert_matmul(offs_flat, x_sorted, w_sorted, W, b)
    output = _combine(y_sorted, pos_flat)
    return (output, logits)
